# baseline (speedup 1.0000x reference)
import functools

import jax
import jax.numpy as jnp
from jax import lax
from jax.experimental import pallas as pl
from jax.experimental.pallas import tpu as pltpu
from jax.experimental.pallas import tpu_sc as plsc

_FPS_NUM = 512
_GROUP_NUM = 32
_RADIUS = 0.2
_BQ_BLOCK = 128


def _fps_body(x_ref, y_ref, z_ref, c0x_ref, c0y_ref, c0z_ref,
              cx_ref, cy_ref, cz_ref, dist_ref):
    B, N = x_ref.shape
    dist_ref[...] = jnp.full((B, N), 1e10, dtype=jnp.float32)
    iota = jax.lax.broadcasted_iota(jnp.int32, (B, N), 1)
    col = jax.lax.broadcasted_iota(jnp.int32, cx_ref.shape, 1)

    def body(i, carry):
        cx, cy, cz = carry
        cx_ref[...] = jnp.where(col == i, cx, cx_ref[...])
        cy_ref[...] = jnp.where(col == i, cy, cy_ref[...])
        cz_ref[...] = jnp.where(col == i, cz, cz_ref[...])
        dx = x_ref[...] - cx
        dy = y_ref[...] - cy
        dz = z_ref[...] - cz
        dd = dx * dx + dy * dy + dz * dz
        dold = dist_ref[...]
        dnew = jnp.where(dd < dold, dd, dold)
        dist_ref[...] = dnew
        m = jnp.max(dnew, axis=1, keepdims=True)
        tie = jnp.where(dnew == m, iota, N)
        amin = jnp.min(tie, axis=1, keepdims=True)
        em = iota == amin
        ncx = jnp.sum(jnp.where(em, x_ref[...], 0.0), axis=1, keepdims=True)
        ncy = jnp.sum(jnp.where(em, y_ref[...], 0.0), axis=1, keepdims=True)
        ncz = jnp.sum(jnp.where(em, z_ref[...], 0.0), axis=1, keepdims=True)
        return (ncx, ncy, ncz)

    jax.lax.fori_loop(0, _FPS_NUM, body,
                      (c0x_ref[...], c0y_ref[...], c0z_ref[...]))


def _bq_body(x_ref, y_ref, z_ref, cx_ref, cy_ref, cz_ref,
             ix_ref, dist_ref):
    C = cx_ref.shape[1]
    N = x_ref.shape[2]
    X = x_ref[0]
    Y = y_ref[0]
    Z = z_ref[0]
    cx = cx_ref[0]
    cy = cy_ref[0]
    cz = cz_ref[0]
    dx = X - cx
    dy = Y - cy
    dz = Z - cz
    d = dx * dx + dy * dy + dz * dz
    dist_ref[...] = jnp.where(d > _RADIUS ** 2, 1e10, d)
    iota = jax.lax.broadcasted_iota(jnp.int32, (C, N), 1)
    gcol = jax.lax.broadcasted_iota(jnp.int32, ix_ref.shape[1:], 1)

    def body(k, _):
        dcur = dist_ref[...]
        amin = jnp.argmin(dcur, axis=1).astype(jnp.int32)[:, None]
        ix_ref[0] = jnp.where(gcol == k, amin, ix_ref[0])
        dist_ref[...] = jnp.where(iota == amin, jnp.inf, dcur)
        return 0

    jax.lax.fori_loop(0, _GROUP_NUM, body, 0)


def _sc_gather_body(x_ref, y_ref, z_ref, idx_ref, cx_ref, cy_ref, cz_ref,
                    rx_ref, ry_ref, rz_ref,
                    xv, yv, zv, idxv, cxv, cyv, czv, rxv, ryv, rzv):
    NC = 2
    wid = lax.axis_index("s") * NC + lax.axis_index("c")
    b = wid // 4
    CW = 128
    GW = CW * _GROUP_NUM
    pltpu.sync_copy(x_ref.at[b], xv)
    pltpu.sync_copy(y_ref.at[b], yv)
    pltpu.sync_copy(z_ref.at[b], zv)
    pltpu.sync_copy(idx_ref.at[pl.ds(wid * GW, GW)], idxv)
    pltpu.sync_copy(cx_ref.at[pl.ds(wid * CW, CW)], cxv)
    pltpu.sync_copy(cy_ref.at[pl.ds(wid * CW, CW)], cyv)
    pltpu.sync_copy(cz_ref.at[pl.ds(wid * CW, CW)], czv)

    def body(f, _):
        fv = jnp.full((16,), f, dtype=jnp.int32)
        cxb = plsc.load_gather(cxv, [fv])
        cyb = plsc.load_gather(cyv, [fv])
        czb = plsc.load_gather(czv, [fv])
        for h in range(_GROUP_NUM // 16):
            o = f * _GROUP_NUM + h * 16
            iv = idxv[pl.ds(o, 16)]
            rxv[pl.ds(o, 16)] = plsc.load_gather(xv, [iv]) - cxb
            ryv[pl.ds(o, 16)] = plsc.load_gather(yv, [iv]) - cyb
            rzv[pl.ds(o, 16)] = plsc.load_gather(zv, [iv]) - czb
        return 0

    lax.fori_loop(0, CW, body, 0)
    pltpu.sync_copy(rxv, rx_ref.at[pl.ds(wid * GW, GW)])
    pltpu.sync_copy(ryv, ry_ref.at[pl.ds(wid * GW, GW)])
    pltpu.sync_copy(rzv, rz_ref.at[pl.ds(wid * GW, GW)])


@jax.jit
def kernel(xyz):
    B, N, _ = xyz.shape
    F, G, C = _FPS_NUM, _GROUP_NUM, _BQ_BLOCK
    xyz_t = jnp.transpose(xyz, (2, 0, 1))
    X, Y, Z = xyz_t[0], xyz_t[1], xyz_t[2]

    f0 = jax.random.randint(jax.random.key(1), (B,), 0, N, dtype=jnp.int32)
    c0 = xyz[jnp.arange(B), f0]
    c0x, c0y, c0z = c0[:, 0:1], c0[:, 1:2], c0[:, 2:3]

    cxs = jax.ShapeDtypeStruct((B, F), jnp.float32)
    CX, CY, CZ = pl.pallas_call(
        _fps_body,
        out_shape=(cxs, cxs, cxs),
        scratch_shapes=[pltpu.VMEM((B, N), jnp.float32)],
    )(X, Y, Z, c0x, c0y, c0z)

    cent3 = (CX.reshape(B, F, 1), CY.reshape(B, F, 1), CZ.reshape(B, F, 1))
    X3, Y3, Z3 = (a.reshape(B, 1, N) for a in (X, Y, Z))
    rowspec = pl.BlockSpec((1, 1, N), lambda b, j: (b, 0, 0))
    centspec = pl.BlockSpec((1, C, 1), lambda b, j: (b, j, 0))
    outspec = pl.BlockSpec((1, C, G), lambda b, j: (b, j, 0))
    IX = pl.pallas_call(
        _bq_body,
        grid=(B, F // C),
        in_specs=[rowspec, rowspec, rowspec, centspec, centspec, centspec],
        out_specs=outspec,
        out_shape=jax.ShapeDtypeStruct((B, F, G), jnp.int32),
        scratch_shapes=[pltpu.VMEM((C, N), jnp.float32)],
    )(X3, Y3, Z3, *cent3)

    mesh = plsc.VectorSubcoreMesh(core_axis_name="c", subcore_axis_name="s")
    GW = (F // 4) * G
    relf = jax.ShapeDtypeStruct((B * F * G,), jnp.float32)
    RX, RY, RZ = pl.kernel(
        _sc_gather_body,
        mesh=mesh,
        out_type=(relf, relf, relf),
        compiler_params=pltpu.CompilerParams(needs_layout_passes=False),
        scratch_types=[
            pltpu.VMEM((N,), jnp.float32),
            pltpu.VMEM((N,), jnp.float32),
            pltpu.VMEM((N,), jnp.float32),
            pltpu.VMEM((GW,), jnp.int32),
            pltpu.VMEM((F // 4,), jnp.float32),
            pltpu.VMEM((F // 4,), jnp.float32),
            pltpu.VMEM((F // 4,), jnp.float32),
            pltpu.VMEM((GW,), jnp.float32),
            pltpu.VMEM((GW,), jnp.float32),
            pltpu.VMEM((GW,), jnp.float32),
        ],
    )(X, Y, Z, IX.reshape(-1), CX.reshape(-1), CY.reshape(-1), CZ.reshape(-1))

    cent = jnp.stack([CX, CY, CZ], axis=-1)
    rel = jnp.stack([RX.reshape(B, F, G), RY.reshape(B, F, G),
                     RZ.reshape(B, F, G)], axis=-1)
    combined = jnp.concatenate([cent[:, :, None, :], rel], axis=2)
    return (combined, cent)


# speedup vs baseline: 1.1553x; 1.1553x over previous
import functools

import jax
import jax.numpy as jnp
from jax import lax
from jax.experimental import pallas as pl
from jax.experimental.pallas import tpu as pltpu
from jax.experimental.pallas import tpu_sc as plsc

_FPS_NUM = 512
_GROUP_NUM = 32
_RADIUS = 0.2
_BQ_BLOCK = 128


def _fps_body(x_ref, y_ref, z_ref, c0x_ref, c0y_ref, c0z_ref,
              cx_ref, cy_ref, cz_ref, dist_ref):
    B, N = x_ref.shape
    dist_ref[...] = jnp.full((B, N), 1e10, dtype=jnp.float32)
    iota = jax.lax.broadcasted_iota(jnp.int32, (B, N), 1)
    col = jax.lax.broadcasted_iota(jnp.int32, cx_ref.shape, 1)

    def body(i, carry):
        cx, cy, cz = carry
        cx_ref[...] = jnp.where(col == i, cx, cx_ref[...])
        cy_ref[...] = jnp.where(col == i, cy, cy_ref[...])
        cz_ref[...] = jnp.where(col == i, cz, cz_ref[...])
        dx = x_ref[...] - cx
        dy = y_ref[...] - cy
        dz = z_ref[...] - cz
        dd = dx * dx + dy * dy + dz * dz
        dold = dist_ref[...]
        dnew = jnp.where(dd < dold, dd, dold)
        dist_ref[...] = dnew
        m = jnp.max(dnew, axis=1, keepdims=True)
        tie = jnp.where(dnew == m, iota, N)
        amin = jnp.min(tie, axis=1, keepdims=True)
        em = iota == amin
        ncx = jnp.sum(jnp.where(em, x_ref[...], 0.0), axis=1, keepdims=True)
        ncy = jnp.sum(jnp.where(em, y_ref[...], 0.0), axis=1, keepdims=True)
        ncz = jnp.sum(jnp.where(em, z_ref[...], 0.0), axis=1, keepdims=True)
        return (ncx, ncy, ncz)

    jax.lax.fori_loop(0, _FPS_NUM, body,
                      (c0x_ref[...], c0y_ref[...], c0z_ref[...]))


def _bq_body(x_ref, y_ref, z_ref, cx_ref, cy_ref, cz_ref,
             ix_ref, dist_ref):
    C = cx_ref.shape[1]
    N = x_ref.shape[2]
    X = x_ref[0]
    Y = y_ref[0]
    Z = z_ref[0]
    cx = cx_ref[0]
    cy = cy_ref[0]
    cz = cz_ref[0]
    dx = X - cx
    dy = Y - cy
    dz = Z - cz
    d = dx * dx + dy * dy + dz * dz
    dist_ref[...] = jnp.where(d > _RADIUS ** 2, 1e10, d)
    iota = jax.lax.broadcasted_iota(jnp.int32, (C, N), 1)
    gcol = jax.lax.broadcasted_iota(jnp.int32, ix_ref.shape[1:], 1)

    def body(k, amin_prev):
        dcur = jnp.where(iota == amin_prev, jnp.inf, dist_ref[...])
        dist_ref[...] = dcur
        m = jnp.min(dcur, axis=1, keepdims=True)
        tie = jnp.where(dcur == m, iota, N)
        amin = jnp.min(tie, axis=1, keepdims=True)
        ix_ref[0] = jnp.where(gcol == k, amin, ix_ref[0])
        return amin

    jax.lax.fori_loop(0, _GROUP_NUM, body,
                      jnp.full((C, 1), -1, dtype=jnp.int32))


def _sc_gather_body(x_ref, y_ref, z_ref, idx_ref, cx_ref, cy_ref, cz_ref,
                    rx_ref, ry_ref, rz_ref,
                    xv, yv, zv, idxv, cxv, cyv, czv, rxv, ryv, rzv):
    NC = 2
    wid = lax.axis_index("s") * NC + lax.axis_index("c")
    b = wid // 4
    CW = 128
    GW = CW * _GROUP_NUM
    pltpu.sync_copy(x_ref.at[b], xv)
    pltpu.sync_copy(y_ref.at[b], yv)
    pltpu.sync_copy(z_ref.at[b], zv)
    pltpu.sync_copy(idx_ref.at[pl.ds(wid * GW, GW)], idxv)
    pltpu.sync_copy(cx_ref.at[pl.ds(wid * CW, CW)], cxv)
    pltpu.sync_copy(cy_ref.at[pl.ds(wid * CW, CW)], cyv)
    pltpu.sync_copy(cz_ref.at[pl.ds(wid * CW, CW)], czv)

    def body(f, _):
        fv = jnp.full((16,), f, dtype=jnp.int32)
        cxb = plsc.load_gather(cxv, [fv])
        cyb = plsc.load_gather(cyv, [fv])
        czb = plsc.load_gather(czv, [fv])
        for h in range(_GROUP_NUM // 16):
            o = f * _GROUP_NUM + h * 16
            iv = idxv[pl.ds(o, 16)]
            rxv[pl.ds(o, 16)] = plsc.load_gather(xv, [iv]) - cxb
            ryv[pl.ds(o, 16)] = plsc.load_gather(yv, [iv]) - cyb
            rzv[pl.ds(o, 16)] = plsc.load_gather(zv, [iv]) - czb
        return 0

    lax.fori_loop(0, CW, body, 0)
    pltpu.sync_copy(rxv, rx_ref.at[pl.ds(wid * GW, GW)])
    pltpu.sync_copy(ryv, ry_ref.at[pl.ds(wid * GW, GW)])
    pltpu.sync_copy(rzv, rz_ref.at[pl.ds(wid * GW, GW)])


@jax.jit
def kernel(xyz):
    B, N, _ = xyz.shape
    F, G, C = _FPS_NUM, _GROUP_NUM, _BQ_BLOCK
    xyz_t = jnp.transpose(xyz, (2, 0, 1))
    X, Y, Z = xyz_t[0], xyz_t[1], xyz_t[2]

    f0 = jax.random.randint(jax.random.key(1), (B,), 0, N, dtype=jnp.int32)
    c0 = xyz[jnp.arange(B), f0]
    c0x, c0y, c0z = c0[:, 0:1], c0[:, 1:2], c0[:, 2:3]

    cxs = jax.ShapeDtypeStruct((B, F), jnp.float32)
    CX, CY, CZ = pl.pallas_call(
        _fps_body,
        out_shape=(cxs, cxs, cxs),
        scratch_shapes=[pltpu.VMEM((B, N), jnp.float32)],
    )(X, Y, Z, c0x, c0y, c0z)

    cent3 = (CX.reshape(B, F, 1), CY.reshape(B, F, 1), CZ.reshape(B, F, 1))
    X3, Y3, Z3 = (a.reshape(B, 1, N) for a in (X, Y, Z))
    rowspec = pl.BlockSpec((1, 1, N), lambda b, j: (b, 0, 0))
    centspec = pl.BlockSpec((1, C, 1), lambda b, j: (b, j, 0))
    outspec = pl.BlockSpec((1, C, G), lambda b, j: (b, j, 0))
    IX = pl.pallas_call(
        _bq_body,
        grid=(B, F // C),
        in_specs=[rowspec, rowspec, rowspec, centspec, centspec, centspec],
        out_specs=outspec,
        out_shape=jax.ShapeDtypeStruct((B, F, G), jnp.int32),
        scratch_shapes=[pltpu.VMEM((C, N), jnp.float32)],
    )(X3, Y3, Z3, *cent3)

    mesh = plsc.VectorSubcoreMesh(core_axis_name="c", subcore_axis_name="s")
    GW = (F // 4) * G
    relf = jax.ShapeDtypeStruct((B * F * G,), jnp.float32)
    RX, RY, RZ = pl.kernel(
        _sc_gather_body,
        mesh=mesh,
        out_type=(relf, relf, relf),
        compiler_params=pltpu.CompilerParams(needs_layout_passes=False),
        scratch_types=[
            pltpu.VMEM((N,), jnp.float32),
            pltpu.VMEM((N,), jnp.float32),
            pltpu.VMEM((N,), jnp.float32),
            pltpu.VMEM((GW,), jnp.int32),
            pltpu.VMEM((F // 4,), jnp.float32),
            pltpu.VMEM((F // 4,), jnp.float32),
            pltpu.VMEM((F // 4,), jnp.float32),
            pltpu.VMEM((GW,), jnp.float32),
            pltpu.VMEM((GW,), jnp.float32),
            pltpu.VMEM((GW,), jnp.float32),
        ],
    )(X, Y, Z, IX.reshape(-1), CX.reshape(-1), CY.reshape(-1), CZ.reshape(-1))

    cent = jnp.stack([CX, CY, CZ], axis=-1)
    rel = jnp.stack([RX.reshape(B, F, G), RY.reshape(B, F, G),
                     RZ.reshape(B, F, G)], axis=-1)
    combined = jnp.concatenate([cent[:, :, None, :], rel], axis=2)
    return (combined, cent)
